# baseline (speedup 1.0000x reference)
import dataclasses
import functools

import jax
import jax.numpy as jnp
from jax import lax
from jax.experimental import pallas as pl
from jax.experimental.pallas import tpu as pltpu
from jax.experimental.pallas import tpu_sc as plsc

N_EXP = 8
N_TOK = 4096
D = 1024
F = 4096

ROWS_128 = N_TOK // 128


def _rank_body(e_ref, pos_ref):
    ev = e_ref[...]
    r128 = lax.broadcasted_iota(jnp.int32, (128, 128), 0)
    c128 = lax.broadcasted_iota(jnp.int32, (128, 128), 1)
    upper = (r128 < c128).astype(jnp.float32)
    r32 = lax.broadcasted_iota(jnp.int32, (32, 32), 0)
    c32 = lax.broadcasted_iota(jnp.int32, (32, 32), 1)
    lower = (c32 < r32).astype(jnp.float32)

    pos = jnp.zeros((ROWS_128, 128), jnp.float32)
    off = jnp.float32(0.0)
    for j in range(N_EXP):
        oh = (ev == j).astype(jnp.float32)
        ex_lane = jnp.dot(oh, upper, preferred_element_type=jnp.float32)
        prev_rows = jnp.dot(lower, oh, preferred_element_type=jnp.float32)
        carry = jnp.sum(prev_rows, axis=1, keepdims=True)
        pos = pos + oh * (off + ex_lane + carry)
        off = off + jnp.sum(oh)
    pos_ref[...] = pos.astype(jnp.int32)


def _compute_pos(dispatch_order):
    e2d = dispatch_order.astype(jnp.int32).reshape(ROWS_128, 128)
    pos2d = pl.pallas_call(
        _rank_body,
        out_shape=jax.ShapeDtypeStruct((ROWS_128, 128), jnp.int32),
    )(e2d)
    return pos2d.reshape(N_TOK)


BM = 1024
BF = 2048


def _mlp_body(x_ref, w1_ref, b1_ref, w2_ref, b2_ref, o_ref):
    j = pl.program_id(1)
    h = jnp.dot(x_ref[...], w1_ref[...], preferred_element_type=jnp.float32)
    h = jnp.maximum(h + b1_ref[...], 0.0)
    contrib = jnp.dot(h, w2_ref[...], preferred_element_type=jnp.float32)

    @pl.when(j == 0)
    def _():
        o_ref[...] = contrib + b2_ref[...]

    @pl.when(j != 0)
    def _():
        o_ref[...] += contrib


def _mlp(x, w1, b1, w2, b2):
    return pl.pallas_call(
        _mlp_body,
        grid=(N_TOK // BM, F // BF),
        in_specs=[
            pl.BlockSpec((BM, D), lambda i, j: (i, 0)),
            pl.BlockSpec((D, BF), lambda i, j: (0, j)),
            pl.BlockSpec((1, BF), lambda i, j: (0, j)),
            pl.BlockSpec((BF, D), lambda i, j: (j, 0)),
            pl.BlockSpec((1, D), lambda i, j: (0, 0)),
        ],
        out_specs=pl.BlockSpec((BM, D), lambda i, j: (i, 0)),
        out_shape=jax.ShapeDtypeStruct((N_TOK, D), jnp.float32),
        compiler_params=pltpu.CompilerParams(
            dimension_semantics=("parallel", "arbitrary")
        ),
    )(x, w1, b1.reshape(1, F), w2, b2.reshape(1, D))


B_PER_W = 128
CHUNK = 32


def _sc_compiler_params():
    cp = pltpu.CompilerParams()
    if "needs_layout_passes" in pltpu.CompilerParams.__dataclass_fields__:
        cp = dataclasses.replace(cp, needs_layout_passes=False)
    return cp


def _permute_rows(y, pos):
    mesh = plsc.VectorSubcoreMesh(core_axis_name="c", subcore_axis_name="s")

    @functools.partial(
        pl.kernel,
        mesh=mesh,
        out_type=jax.ShapeDtypeStruct((N_TOK, D), jnp.float32),
        scratch_types=[
            pltpu.VMEM((CHUNK,), jnp.int32),
            pltpu.VMEM((CHUNK,), jnp.int32),
            pltpu.VMEM((CHUNK, D), jnp.float32),
            pltpu.VMEM((CHUNK, D), jnp.float32),
            pltpu.SemaphoreType.DMA,
            pltpu.SemaphoreType.DMA,
            pltpu.SemaphoreType.DMA,
            pltpu.SemaphoreType.DMA,
        ],
        compiler_params=_sc_compiler_params(),
    )
    def permute_kernel(
        y_hbm, pos_hbm, out_hbm, idx0, idx1, rows0, rows1, s0, s1, s2, s3
    ):
        wid = lax.axis_index("s") * 2 + lax.axis_index("c")
        base = wid * B_PER_W
        idx = (idx0, idx1)
        rows = (rows0, rows1)
        lsem = (s0, s1)
        ssem = (s2, s3)

        def load(c):
            off = base + c * CHUNK
            p = pltpu.async_copy(pos_hbm.at[pl.ds(off, CHUNK)], idx[c % 2], lsem[c % 2])
            r = pltpu.async_copy(y_hbm.at[pl.ds(off, CHUNK)], rows[c % 2], lsem[c % 2])
            return p, r

        def scatter(c):
            b = c % 2
            return pltpu.async_copy(rows[b], out_hbm.at[idx[b]], ssem[b])

        def wait2(pr):
            pr[0].wait()
            pr[1].wait()

        l0, l1 = load(0), load(1)
        wait2(l0)
        sc0 = scatter(0)
        wait2(l1)
        sc1 = scatter(1)
        sc0.wait()
        l2 = load(2)
        sc1.wait()
        l3 = load(3)
        wait2(l2)
        sc2 = scatter(2)
        wait2(l3)
        sc3 = scatter(3)
        sc2.wait()
        sc3.wait()

    return permute_kernel(y, pos)


def kernel(inputs, dispatch_order, W1, b1, W2, b2):
    B, S, Dm = inputs.shape
    flat = inputs.reshape(B * S, Dm)
    pos = _compute_pos(dispatch_order)
    y = _mlp(flat, W1, b1, W2, b2)
    return _permute_rows(y, pos)


# speedup vs baseline: 1.0018x; 1.0018x over previous
import dataclasses
import functools

import jax
import jax.numpy as jnp
from jax import lax
from jax.experimental import pallas as pl
from jax.experimental.pallas import tpu as pltpu
from jax.experimental.pallas import tpu_sc as plsc

N_EXP = 8
N_TOK = 4096
D = 1024
F = 4096

ROWS_128 = N_TOK // 128


def _rank_body(e_ref, pos_ref):
    ev = e_ref[...].reshape(ROWS_128, 128)
    r128 = lax.broadcasted_iota(jnp.int32, (128, 128), 0)
    c128 = lax.broadcasted_iota(jnp.int32, (128, 128), 1)
    upper = (r128 < c128).astype(jnp.float32)
    r32 = lax.broadcasted_iota(jnp.int32, (32, 32), 0)
    c32 = lax.broadcasted_iota(jnp.int32, (32, 32), 1)
    lower = (c32 < r32).astype(jnp.float32)

    pos = jnp.zeros((ROWS_128, 128), jnp.float32)
    off = jnp.float32(0.0)
    for j in range(N_EXP):
        oh = (ev == j).astype(jnp.float32)
        ex_lane = jnp.dot(oh, upper, preferred_element_type=jnp.float32)
        prev_rows = jnp.dot(lower, oh, preferred_element_type=jnp.float32)
        carry = jnp.sum(prev_rows, axis=1, keepdims=True)
        pos = pos + oh * (off + ex_lane + carry)
        off = off + jnp.sum(oh)
    pos_ref[...] = pos.astype(jnp.int32).reshape(N_TOK)


def _compute_pos(dispatch_order):
    return pl.pallas_call(
        _rank_body,
        out_shape=jax.ShapeDtypeStruct((N_TOK,), jnp.int32),
    )(dispatch_order.astype(jnp.int32))


BM = 1024
BF = 2048


def _mlp_body(x_ref, w1_ref, b1_ref, w2_ref, b2_ref, o_ref):
    j = pl.program_id(1)
    h = jnp.dot(x_ref[...], w1_ref[...], preferred_element_type=jnp.float32)
    h = jnp.maximum(h + b1_ref[...], 0.0)
    contrib = jnp.dot(h, w2_ref[...], preferred_element_type=jnp.float32)

    @pl.when(j == 0)
    def _():
        o_ref[...] = contrib + b2_ref[...]

    @pl.when(j != 0)
    def _():
        o_ref[...] += contrib


def _mlp(x, w1, b1, w2, b2):
    return pl.pallas_call(
        _mlp_body,
        grid=(N_TOK // BM, F // BF),
        in_specs=[
            pl.BlockSpec((BM, D), lambda i, j: (i, 0)),
            pl.BlockSpec((D, BF), lambda i, j: (0, j)),
            pl.BlockSpec((1, BF), lambda i, j: (0, j)),
            pl.BlockSpec((BF, D), lambda i, j: (j, 0)),
            pl.BlockSpec((1, D), lambda i, j: (0, 0)),
        ],
        out_specs=pl.BlockSpec((BM, D), lambda i, j: (i, 0)),
        out_shape=jax.ShapeDtypeStruct((N_TOK, D), jnp.float32),
        compiler_params=pltpu.CompilerParams(
            dimension_semantics=("parallel", "arbitrary")
        ),
    )(x, w1, b1.reshape(1, F), w2, b2.reshape(1, D))


B_PER_W = 128
CHUNK = 32


def _sc_compiler_params():
    cp = pltpu.CompilerParams()
    if "needs_layout_passes" in pltpu.CompilerParams.__dataclass_fields__:
        cp = dataclasses.replace(cp, needs_layout_passes=False)
    return cp


def _permute_rows(y, pos):
    mesh = plsc.VectorSubcoreMesh(core_axis_name="c", subcore_axis_name="s")

    @functools.partial(
        pl.kernel,
        mesh=mesh,
        out_type=jax.ShapeDtypeStruct((N_TOK, D), jnp.float32),
        scratch_types=[
            pltpu.VMEM((CHUNK,), jnp.int32),
            pltpu.VMEM((CHUNK,), jnp.int32),
            pltpu.VMEM((CHUNK, D), jnp.float32),
            pltpu.VMEM((CHUNK, D), jnp.float32),
            pltpu.SemaphoreType.DMA,
            pltpu.SemaphoreType.DMA,
            pltpu.SemaphoreType.DMA,
            pltpu.SemaphoreType.DMA,
        ],
        compiler_params=_sc_compiler_params(),
    )
    def permute_kernel(
        y_hbm, pos_hbm, out_hbm, idx0, idx1, rows0, rows1, s0, s1, s2, s3
    ):
        wid = lax.axis_index("s") * 2 + lax.axis_index("c")
        base = wid * B_PER_W
        idx = (idx0, idx1)
        rows = (rows0, rows1)
        lsem = (s0, s1)
        ssem = (s2, s3)

        def load(c):
            off = base + c * CHUNK
            p = pltpu.async_copy(pos_hbm.at[pl.ds(off, CHUNK)], idx[c % 2], lsem[c % 2])
            r = pltpu.async_copy(y_hbm.at[pl.ds(off, CHUNK)], rows[c % 2], lsem[c % 2])
            return p, r

        def scatter(c):
            b = c % 2
            return pltpu.async_copy(rows[b], out_hbm.at[idx[b]], ssem[b])

        def wait2(pr):
            pr[0].wait()
            pr[1].wait()

        l0, l1 = load(0), load(1)
        wait2(l0)
        sc0 = scatter(0)
        wait2(l1)
        sc1 = scatter(1)
        sc0.wait()
        l2 = load(2)
        sc1.wait()
        l3 = load(3)
        wait2(l2)
        sc2 = scatter(2)
        wait2(l3)
        sc3 = scatter(3)
        sc2.wait()
        sc3.wait()

    return permute_kernel(y, pos)


def kernel(inputs, dispatch_order, W1, b1, W2, b2):
    B, S, Dm = inputs.shape
    flat = inputs.reshape(B * S, Dm)
    pos = _compute_pos(dispatch_order)
    y = _mlp(flat, W1, b1, W2, b2)
    return _permute_rows(y, pos)
